# Initial kernel scaffold; baseline (speedup 1.0000x reference)
import jax
import jax.numpy as jnp
from jax.experimental import pallas as pl


def kernel(p1, p2, f2, W, gamma, beta):
    raise NotImplementedError("write your pallas kernel here")


# speedup vs baseline: 38.7933x; 38.7933x over previous
import functools

import jax
import jax.numpy as jnp
from jax import lax
from jax.experimental import pallas as pl

B, N1, N2, C_IN, C_OUT = 4, 8192, 2048, 256, 64
TILE = 512


def _conv_bn_relu_kernel(f2_ref, w_ref, gamma_ref, beta_ref, f_ref):
    s = jnp.zeros((C_OUT, 1), jnp.float32)
    ss = jnp.zeros((C_OUT, 1), jnp.float32)
    for b in range(B):
        g = lax.dot_general(
            w_ref[...], f2_ref[b],
            (((1,), (0,)), ((), ())),
            preferred_element_type=jnp.float32,
        )
        s = s + jnp.sum(g, axis=1, keepdims=True)
        ss = ss + jnp.sum(g * g, axis=1, keepdims=True)
        f_ref[b] = g
    cnt = float(B * N2)
    mean = s / cnt
    var = ss / cnt - mean * mean
    scale = gamma_ref[...] * lax.rsqrt(var + 1e-5)
    shift = beta_ref[...] - mean * scale
    for b in range(B):
        f_ref[b] = jnp.maximum(f_ref[b] * scale + shift, 0.0)


def _top3_interp_kernel(p1_ref, p2_ref, f_ref, out_ref):
    p1t = p1_ref[0]
    p2t = p2_ref[0]
    dots = lax.dot_general(
        p1t, p2t, (((1,), (1,)), ((), ())),
        preferred_element_type=jnp.float32,
    )
    n1sq = jnp.sum(p1t * p1t, axis=1)[:, None]
    n2sq = jnp.sum(p2t * p2t, axis=1)[None, :]
    d2 = n1sq + n2sq - 2.0 * dots

    iota = lax.broadcasted_iota(jnp.int32, (TILE, N2), 1)
    ms = []
    js = []
    for k in range(3):
        mk = jnp.min(d2, axis=1)
        jk = jnp.min(jnp.where(d2 == mk[:, None], iota, N2), axis=1)
        ms.append(mk)
        js.append(jk)
        if k < 2:
            d2 = jnp.where(iota == jk[:, None], jnp.inf, d2)

    recips = [1.0 / (jnp.maximum(m, 0.0) + 1e-8) for m in ms]
    norm = recips[0] + recips[1] + recips[2]
    ws = [r / norm for r in recips]

    iota0 = lax.broadcasted_iota(jnp.int32, (N2, TILE), 0)
    s_t = jnp.zeros((N2, TILE), jnp.float32)
    for k in range(3):
        s_t = s_t + jnp.where(iota0 == js[k][None, :], ws[k][None, :], 0.0)

    out_ref[0] = lax.dot_general(
        f_ref[0], s_t, (((1,), (0,)), ((), ())),
        preferred_element_type=jnp.float32,
    )


def kernel(p1, p2, f2, W, gamma, beta):
    f = pl.pallas_call(
        _conv_bn_relu_kernel,
        out_shape=jax.ShapeDtypeStruct((B, C_OUT, N2), jnp.float32),
    )(f2, W, gamma.reshape(C_OUT, 1), beta.reshape(C_OUT, 1))

    out = pl.pallas_call(
        _top3_interp_kernel,
        grid=(B, N1 // TILE),
        in_specs=[
            pl.BlockSpec((1, TILE, 3), lambda b, t: (b, t, 0)),
            pl.BlockSpec((1, N2, 3), lambda b, t: (b, 0, 0)),
            pl.BlockSpec((1, C_OUT, N2), lambda b, t: (b, 0, 0)),
        ],
        out_specs=pl.BlockSpec((1, C_OUT, TILE), lambda b, t: (b, 0, t)),
        out_shape=jax.ShapeDtypeStruct((B, C_OUT, N1), jnp.float32),
    )(p1, p2, f)
    return out
